# baseline (speedup 1.0000x reference)
import functools

import jax
import jax.numpy as jnp
from jax import lax
from jax.experimental import pallas as pl
from jax.experimental.pallas import tpu as pltpu
from jax.experimental.pallas import tpu_sc as plsc

N_NODES = 10000
N_EDGES = 320000
D_IN = 128
D_HID = 128
N_CLASSES = 47

NC = 2
NS = 16
NW = NC * NS
CHUNK = 128
N_CHUNKS = 80
E_PAD = NW * CHUNK * N_CHUNKS
NPAD = 10240
ROWS_PER_TILE = NPAD // NS
D2 = 48

BLK = 2000
NB = N_NODES // BLK


def _make_sc_agg(d, with_deg):
  mesh = plsc.VectorSubcoreMesh(core_axis_name="c", subcore_axis_name="s")
  out_type = [jax.ShapeDtypeStruct((NC, NPAD, d), jnp.float32)]
  scratch = [
      pltpu.VMEM((N_CHUNKS // 2, CHUNK), jnp.int32),
      pltpu.VMEM((N_CHUNKS // 2, CHUNK), jnp.int32),
      pltpu.VMEM((CHUNK, d), jnp.float32),
      pltpu.VMEM((CHUNK, d), jnp.float32),
      pltpu.VMEM_SHARED((NPAD, d), jnp.float32),
      pltpu.SemaphoreType.DMA,
      pltpu.SemaphoreType.DMA,
      pltpu.SemaphoreType.DMA,
      pltpu.SemaphoreType.DMA,
  ]
  if with_deg:
    out_type.append(jax.ShapeDtypeStruct((NC, NPAD), jnp.float32))
    scratch += [
        pltpu.VMEM((CHUNK,), jnp.float32),
        pltpu.VMEM_SHARED((NPAD,), jnp.float32),
        pltpu.SemaphoreType.DMA,
    ]

  def body(table_hbm, zeros2d_hbm, zeros1d_hbm, ones_hbm, src_hbm, dst_hbm,
           part_hbm, *rest):
    if with_deg:
      (deg_hbm, src_v, dst_v, rows0_v, rows1_v, acc_sh, gsem0, gsem1,
       ssem0, ssem1, ones_v, deg_sh, dsem) = rest
    else:
      (src_v, dst_v, rows0_v, rows1_v, acc_sh, gsem0, gsem1,
       ssem0, ssem1) = rest
    bufs = ((rows0_v, gsem0, ssem0), (rows1_v, gsem1, ssem1))
    c = lax.axis_index("c")
    s = lax.axis_index("s")
    w = c * NS + s
    r0 = s * ROWS_PER_TILE

    if with_deg:
      pltpu.sync_copy(ones_hbm, ones_v)
      pltpu.sync_copy(zeros1d_hbm.at[pl.ds(r0, ROWS_PER_TILE)],
                      deg_sh.at[pl.ds(r0, ROWS_PER_TILE)])
    pltpu.sync_copy(zeros2d_hbm.at[pl.ds(r0, ROWS_PER_TILE)],
                    acc_sh.at[pl.ds(r0, ROWS_PER_TILE)])
    plsc.subcore_barrier()

    half = N_CHUNKS // 2
    for p in range(2):
      pltpu.sync_copy(src_hbm.at[w, pl.ds(p * half, half)], src_v)
      pltpu.sync_copy(dst_hbm.at[w, pl.ds(p * half, half)], dst_v)
      pltpu.make_async_copy(table_hbm.at[src_v.at[0]], rows0_v,
                            gsem0).start()

      pltpu.make_async_copy(table_hbm.at[src_v.at[1]], rows1_v,
                            gsem1).start()

      @pl.loop(0, half, step=2)
      def _(j):
        j1 = lax.rem(j + 1, half)
        j2 = lax.rem(j + 2, half)
        j3 = lax.rem(j + 3, half)
        pltpu.make_async_copy(table_hbm.at[src_v.at[j]], rows0_v,
                              gsem0).wait()
        pltpu.make_async_copy(rows0_v, acc_sh.at[dst_v.at[j]],
                              ssem0).start(add=True)
        if with_deg:
          pltpu.make_async_copy(ones_v, deg_sh.at[dst_v.at[j]],
                                dsem).start(add=True)
        pltpu.make_async_copy(table_hbm.at[src_v.at[j1]], rows1_v,
                              gsem1).wait()
        pltpu.make_async_copy(rows1_v, acc_sh.at[dst_v.at[j1]],
                              ssem1).start(add=True)
        if with_deg:
          pltpu.make_async_copy(ones_v, deg_sh.at[dst_v.at[j1]],
                                dsem).start(add=True)
        pltpu.make_async_copy(rows0_v, acc_sh.at[dst_v.at[j]], ssem0).wait()
        pltpu.make_async_copy(table_hbm.at[src_v.at[j2]], rows0_v,
                              gsem0).start()
        pltpu.make_async_copy(rows1_v, acc_sh.at[dst_v.at[j1]],
                              ssem1).wait()
        pltpu.make_async_copy(table_hbm.at[src_v.at[j3]], rows1_v,
                              gsem1).start()

      pltpu.make_async_copy(table_hbm.at[src_v.at[0]], rows0_v, gsem0).wait()
      pltpu.make_async_copy(table_hbm.at[src_v.at[1]], rows1_v, gsem1).wait()
      if with_deg:
        @pl.loop(0, half)
        def _(j):
          pltpu.make_async_copy(ones_v, deg_sh.at[dst_v.at[0]], dsem).wait()
    plsc.subcore_barrier()
    pltpu.sync_copy(acc_sh.at[pl.ds(r0, ROWS_PER_TILE)],
                    part_hbm.at[c, pl.ds(r0, ROWS_PER_TILE)])
    if with_deg:
      pltpu.sync_copy(deg_sh.at[pl.ds(r0, ROWS_PER_TILE)],
                      deg_hbm.at[c, pl.ds(r0, ROWS_PER_TILE)])

  params = None if d % 128 == 0 else pltpu.CompilerParams(
      use_tc_tiling_on_sc=False)
  return pl.kernel(body, out_type=out_type, mesh=mesh, scratch_types=scratch,
                   compiler_params=params, name=f"sc_segsum_d{d}")


_sc_agg_l0 = _make_sc_agg(D_IN, with_deg=True)
_sc_agg_l1 = _make_sc_agg(D2, with_deg=False)


def _tc1_body(x_ref, p_ref, degp_ref, ws0_ref, wn0_ref, b0_ref, wn1_ref,
              h_ref, z_ref):
  deg = jnp.maximum(degp_ref[0, :, 0] + degp_ref[1, :, 0], 1.0)
  hn = (p_ref[0] + p_ref[1]) / deg[:, None]
  h = (jnp.dot(x_ref[...], ws0_ref[...], preferred_element_type=jnp.float32)
       + jnp.dot(hn, wn0_ref[...], preferred_element_type=jnp.float32)
       + b0_ref[...])
  h = jnp.maximum(h, 0.0)
  h_ref[...] = h
  z_ref[...] = jnp.dot(h, wn1_ref[...], preferred_element_type=jnp.float32)


def _tc2_body(h_ref, q_ref, degp_ref, ws1_ref, b1_ref, out_ref):
  deg = jnp.maximum(degp_ref[0, :, 0] + degp_ref[1, :, 0], 1.0)
  hn = (q_ref[0] + q_ref[1]) / deg[:, None]
  out_ref[...] = (
      jnp.dot(h_ref[...], ws1_ref[...], preferred_element_type=jnp.float32)
      + hn + b1_ref[...])


_tc1 = pl.pallas_call(
    _tc1_body,
    grid=(NB,),
    in_specs=[
        pl.BlockSpec((BLK, D_IN), lambda i: (i, 0)),
        pl.BlockSpec((NC, BLK, D_IN), lambda i: (0, i, 0)),
        pl.BlockSpec((NC, BLK, 1), lambda i: (0, i, 0)),
        pl.BlockSpec((D_IN, D_HID), lambda i: (0, 0)),
        pl.BlockSpec((D_IN, D_HID), lambda i: (0, 0)),
        pl.BlockSpec((1, D_HID), lambda i: (0, 0)),
        pl.BlockSpec((D_HID, D2), lambda i: (0, 0)),
    ],
    out_specs=[
        pl.BlockSpec((BLK, D_HID), lambda i: (i, 0)),
        pl.BlockSpec((BLK, D2), lambda i: (i, 0)),
    ],
    out_shape=[
        jax.ShapeDtypeStruct((N_NODES, D_HID), jnp.float32),
        jax.ShapeDtypeStruct((N_NODES, D2), jnp.float32),
    ],
)

_tc2 = pl.pallas_call(
    _tc2_body,
    grid=(NB,),
    in_specs=[
        pl.BlockSpec((BLK, D_HID), lambda i: (i, 0)),
        pl.BlockSpec((NC, BLK, D2), lambda i: (0, i, 0)),
        pl.BlockSpec((NC, BLK, 1), lambda i: (0, i, 0)),
        pl.BlockSpec((D_HID, D2), lambda i: (0, 0)),
        pl.BlockSpec((1, D2), lambda i: (0, 0)),
    ],
    out_specs=pl.BlockSpec((BLK, D2), lambda i: (i, 0)),
    out_shape=jax.ShapeDtypeStruct((N_NODES, D2), jnp.float32),
)


@jax.jit
def kernel(x, edge_index, W_self0, W_neigh0, b0, W_self1, W_neigh1, b1):
  src = edge_index[0].astype(jnp.int32)
  dst = edge_index[1].astype(jnp.int32)
  pad = E_PAD - N_EDGES
  ppw = pad // NW
  ar = jnp.arange(pad, dtype=jnp.int32)
  pad_src = ar % N_NODES
  pad_dst = N_NODES + ar % (NPAD - N_NODES)
  srcp = jnp.concatenate([src.reshape(NW, -1), pad_src.reshape(NW, ppw)], 1)
  dstp = jnp.concatenate([dst.reshape(NW, -1), pad_dst.reshape(NW, ppw)], 1)
  srcp = srcp.reshape(NW, N_CHUNKS, CHUNK)
  dstp = dstp.reshape(NW, N_CHUNKS, CHUNK)

  zeros2d = jnp.zeros((NPAD, D_IN), jnp.float32)
  zeros2d_s = jnp.zeros((NPAD, D2), jnp.float32)
  zeros1d = jnp.zeros((NPAD,), jnp.float32)
  ones = jnp.ones((CHUNK,), jnp.float32)

  part0, degp = _sc_agg_l0(x, zeros2d, zeros1d, ones, srcp, dstp)

  b0r = b0.reshape(1, D_HID)
  wn1p = jnp.pad(W_neigh1, ((0, 0), (0, D2 - N_CLASSES)))
  degp3 = degp.reshape(NC, NPAD, 1)
  h, z = _tc1(x, part0, degp3, W_self0, W_neigh0, b0r, wn1p)

  (part1,) = (_sc_agg_l1(z, zeros2d_s, zeros1d, ones, srcp, dstp),)
  part1 = part1[0] if isinstance(part1, (list, tuple)) else part1

  ws1p = jnp.pad(W_self1, ((0, 0), (0, D2 - N_CLASSES)))
  b1p = jnp.pad(b1, (0, D2 - N_CLASSES)).reshape(1, D2)
  out = _tc2(h, part1, degp3, ws1p, b1p)
  return out[:, :N_CLASSES]


# speedup vs baseline: 1.0229x; 1.0229x over previous
import functools

import jax
import jax.numpy as jnp
import numpy as np
from jax import lax
from jax.experimental import pallas as pl
from jax.experimental.pallas import tpu as pltpu
from jax.experimental.pallas import tpu_sc as plsc

N_NODES = 10000
N_EDGES = 320000
D_IN = 128
D_HID = 128
N_CLASSES = 47

NC = 2
NS = 16
NW = NC * NS
CHUNK = 128
N_CHUNKS = 80
E_PAD = NW * CHUNK * N_CHUNKS
NPAD = 10240
ROWS_PER_TILE = NPAD // NS
D2 = 48

_PAD = E_PAD - N_EDGES
_PAD_SRC = (np.arange(_PAD, dtype=np.int32) % N_NODES).reshape(NW, _PAD // NW)
_PAD_DST = (N_NODES + np.arange(_PAD, dtype=np.int32)
            % (NPAD - N_NODES)).reshape(NW, _PAD // NW)

BLK = 2000
NB = N_NODES // BLK


def _make_sc_agg(d, with_deg):
  mesh = plsc.VectorSubcoreMesh(core_axis_name="c", subcore_axis_name="s")
  sup = 2 if d <= 64 else 1
  out_type = [jax.ShapeDtypeStruct((NC, NPAD, d), jnp.float32)]
  scratch = [
      pltpu.VMEM((N_CHUNKS // 2, CHUNK), jnp.int32),
      pltpu.VMEM((N_CHUNKS // 2, CHUNK), jnp.int32),
      pltpu.VMEM((sup * CHUNK, d), jnp.float32),
      pltpu.VMEM((sup * CHUNK, d), jnp.float32),
      pltpu.VMEM_SHARED((NPAD, d), jnp.float32),
      pltpu.SemaphoreType.DMA,
      pltpu.SemaphoreType.DMA,
      pltpu.SemaphoreType.DMA,
      pltpu.SemaphoreType.DMA,
  ]
  if with_deg:
    out_type.append(jax.ShapeDtypeStruct((NC, NPAD), jnp.float32))
    scratch += [
        pltpu.VMEM((CHUNK,), jnp.float32),
        pltpu.VMEM_SHARED((NPAD,), jnp.float32),
        pltpu.SemaphoreType.DMA,
    ]

  def body(table_hbm, zeros2d_hbm, zeros1d_hbm, ones_hbm, src_hbm, dst_hbm,
           part_hbm, *rest):
    if with_deg:
      (deg_hbm, src_v, dst_v, rows0_v, rows1_v, acc_sh, gsem0, gsem1,
       ssem0, ssem1, ones_v, deg_sh, dsem) = rest
    else:
      (src_v, dst_v, rows0_v, rows1_v, acc_sh, gsem0, gsem1,
       ssem0, ssem1) = rest
    bufs = ((rows0_v, gsem0, ssem0), (rows1_v, gsem1, ssem1))
    c = lax.axis_index("c")
    s = lax.axis_index("s")
    w = c * NS + s
    r0 = s * ROWS_PER_TILE

    if with_deg:
      pltpu.sync_copy(ones_hbm, ones_v)
      pltpu.sync_copy(zeros1d_hbm.at[pl.ds(r0, ROWS_PER_TILE)],
                      deg_sh.at[pl.ds(r0, ROWS_PER_TILE)])
    pltpu.sync_copy(zeros2d_hbm.at[pl.ds(r0, ROWS_PER_TILE)],
                    acc_sh.at[pl.ds(r0, ROWS_PER_TILE)])
    plsc.subcore_barrier()

    half = N_CHUNKS // 2
    nsup = half // sup

    def g_start(m, rows_v, gsem):
      for q in range(sup):
        pltpu.make_async_copy(table_hbm.at[src_v.at[sup * m + q]],
                              rows_v.at[pl.ds(q * CHUNK, CHUNK)],
                              gsem).start()

    def g_wait(m, rows_v, gsem):
      for q in range(sup):
        pltpu.make_async_copy(table_hbm.at[src_v.at[sup * m + q]],
                              rows_v.at[pl.ds(q * CHUNK, CHUNK)],
                              gsem).wait()

    def s_start(m, rows_v, ssem):
      for q in range(sup):
        pltpu.make_async_copy(rows_v.at[pl.ds(q * CHUNK, CHUNK)],
                              acc_sh.at[dst_v.at[sup * m + q]],
                              ssem).start(add=True)
        if with_deg:
          pltpu.make_async_copy(ones_v, deg_sh.at[dst_v.at[sup * m + q]],
                                dsem).start(add=True)

    def s_wait(m, rows_v, ssem):
      for q in range(sup):
        pltpu.make_async_copy(rows_v.at[pl.ds(q * CHUNK, CHUNK)],
                              acc_sh.at[dst_v.at[sup * m + q]],
                              ssem).wait()

    for p in range(2):
      pltpu.sync_copy(src_hbm.at[w, pl.ds(p * half, half)], src_v)
      pltpu.sync_copy(dst_hbm.at[w, pl.ds(p * half, half)], dst_v)
      g_start(0, rows0_v, gsem0)
      g_start(1, rows1_v, gsem1)

      @pl.loop(0, nsup, step=2)
      def _(j):
        j1 = lax.rem(j + 1, nsup)
        j2 = lax.rem(j + 2, nsup)
        j3 = lax.rem(j + 3, nsup)
        g_wait(j, rows0_v, gsem0)
        s_start(j, rows0_v, ssem0)
        g_wait(j1, rows1_v, gsem1)
        s_start(j1, rows1_v, ssem1)
        s_wait(j, rows0_v, ssem0)
        g_start(j2, rows0_v, gsem0)
        s_wait(j1, rows1_v, ssem1)
        g_start(j3, rows1_v, gsem1)

      g_wait(0, rows0_v, gsem0)
      g_wait(1, rows1_v, gsem1)
      if with_deg:
        @pl.loop(0, half)
        def _(j):
          pltpu.make_async_copy(ones_v, deg_sh.at[dst_v.at[0]], dsem).wait()
    plsc.subcore_barrier()
    pltpu.sync_copy(acc_sh.at[pl.ds(r0, ROWS_PER_TILE)],
                    part_hbm.at[c, pl.ds(r0, ROWS_PER_TILE)])
    if with_deg:
      pltpu.sync_copy(deg_sh.at[pl.ds(r0, ROWS_PER_TILE)],
                      deg_hbm.at[c, pl.ds(r0, ROWS_PER_TILE)])

  params = None if d % 128 == 0 else pltpu.CompilerParams(
      use_tc_tiling_on_sc=False)
  return pl.kernel(body, out_type=out_type, mesh=mesh, scratch_types=scratch,
                   compiler_params=params, name=f"sc_segsum_d{d}")


_sc_agg_l0 = _make_sc_agg(D_IN, with_deg=True)
_sc_agg_l1 = _make_sc_agg(D2, with_deg=False)


def _tc1_body(x_ref, p_ref, degp_ref, ws0_ref, wn0_ref, b0_ref, wn1_ref,
              h_ref, z_ref):
  deg = jnp.maximum(degp_ref[0, :, 0] + degp_ref[1, :, 0], 1.0)
  hn = (p_ref[0] + p_ref[1]) / deg[:, None]
  h = (jnp.dot(x_ref[...], ws0_ref[...], preferred_element_type=jnp.float32)
       + jnp.dot(hn, wn0_ref[...], preferred_element_type=jnp.float32)
       + b0_ref[...])
  h = jnp.maximum(h, 0.0)
  h_ref[...] = h
  z_ref[...] = jnp.dot(h, wn1_ref[...], preferred_element_type=jnp.float32)


def _tc2_body(h_ref, q_ref, degp_ref, ws1_ref, b1_ref, out_ref):
  deg = jnp.maximum(degp_ref[0, :, 0] + degp_ref[1, :, 0], 1.0)
  hn = (q_ref[0] + q_ref[1]) / deg[:, None]
  out_ref[...] = (
      jnp.dot(h_ref[...], ws1_ref[...], preferred_element_type=jnp.float32)
      + hn + b1_ref[...])


_tc1 = pl.pallas_call(
    _tc1_body,
    grid=(NB,),
    in_specs=[
        pl.BlockSpec((BLK, D_IN), lambda i: (i, 0)),
        pl.BlockSpec((NC, BLK, D_IN), lambda i: (0, i, 0)),
        pl.BlockSpec((NC, BLK, 1), lambda i: (0, i, 0)),
        pl.BlockSpec((D_IN, D_HID), lambda i: (0, 0)),
        pl.BlockSpec((D_IN, D_HID), lambda i: (0, 0)),
        pl.BlockSpec((1, D_HID), lambda i: (0, 0)),
        pl.BlockSpec((D_HID, D2), lambda i: (0, 0)),
    ],
    out_specs=[
        pl.BlockSpec((BLK, D_HID), lambda i: (i, 0)),
        pl.BlockSpec((BLK, D2), lambda i: (i, 0)),
    ],
    out_shape=[
        jax.ShapeDtypeStruct((N_NODES, D_HID), jnp.float32),
        jax.ShapeDtypeStruct((N_NODES, D2), jnp.float32),
    ],
)

_tc2 = pl.pallas_call(
    _tc2_body,
    grid=(NB,),
    in_specs=[
        pl.BlockSpec((BLK, D_HID), lambda i: (i, 0)),
        pl.BlockSpec((NC, BLK, D2), lambda i: (0, i, 0)),
        pl.BlockSpec((NC, BLK, 1), lambda i: (0, i, 0)),
        pl.BlockSpec((D_HID, D2), lambda i: (0, 0)),
        pl.BlockSpec((1, D2), lambda i: (0, 0)),
    ],
    out_specs=pl.BlockSpec((BLK, D2), lambda i: (i, 0)),
    out_shape=jax.ShapeDtypeStruct((N_NODES, D2), jnp.float32),
)


@jax.jit
def kernel(x, edge_index, W_self0, W_neigh0, b0, W_self1, W_neigh1, b1):
  src = edge_index[0].astype(jnp.int32)
  dst = edge_index[1].astype(jnp.int32)
  pad = E_PAD - N_EDGES
  ppw = pad // NW
  srcp = jnp.concatenate([src.reshape(NW, -1), _PAD_SRC], 1)
  dstp = jnp.concatenate([dst.reshape(NW, -1), _PAD_DST], 1)
  srcp = srcp.reshape(NW, N_CHUNKS, CHUNK)
  dstp = dstp.reshape(NW, N_CHUNKS, CHUNK)

  zeros2d = jnp.zeros((NPAD, D_IN), jnp.float32)
  zeros2d_s = jnp.zeros((NPAD, D2), jnp.float32)
  zeros1d = jnp.zeros((NPAD,), jnp.float32)
  ones = jnp.ones((CHUNK,), jnp.float32)

  part0, degp = _sc_agg_l0(x, zeros2d, zeros1d, ones, srcp, dstp)

  b0r = b0.reshape(1, D_HID)
  wn1p = jnp.pad(W_neigh1, ((0, 0), (0, D2 - N_CLASSES)))
  degp3 = degp.reshape(NC, NPAD, 1)
  h, z = _tc1(x, part0, degp3, W_self0, W_neigh0, b0r, wn1p)

  (part1,) = (_sc_agg_l1(z, zeros2d_s, zeros1d, ones, srcp, dstp),)
  part1 = part1[0] if isinstance(part1, (list, tuple)) else part1

  ws1p = jnp.pad(W_self1, ((0, 0), (0, D2 - N_CLASSES)))
  b1p = jnp.pad(b1, (0, D2 - N_CLASSES)).reshape(1, D2)
  out = _tc2(h, part1, degp3, ws1p, b1p)
  return out[:, :N_CLASSES]


# speedup vs baseline: 1.0563x; 1.0326x over previous
import functools

import jax
import jax.numpy as jnp
import numpy as np
from jax import lax
from jax.experimental import pallas as pl
from jax.experimental.pallas import tpu as pltpu
from jax.experimental.pallas import tpu_sc as plsc

N_NODES = 10000
N_EDGES = 320000
D_IN = 128
D_HID = 128
N_CLASSES = 47

NC = 2
NS = 16
NW = NC * NS
CHUNK = 128
TOT_CHUNKS = N_EDGES // CHUNK
BASE_CHK = TOT_CHUNKS // NW
XTRA = TOT_CHUNKS - BASE_CHK * NW
NPAD = 10240
ROWS_PER_TILE = NPAD // NS
D2 = 48

BLK = 2000
NB = N_NODES // BLK


def _make_sc_agg(d, with_deg):
  mesh = plsc.VectorSubcoreMesh(core_axis_name="c", subcore_axis_name="s")
  sup = 2 if d <= 64 else 1
  passes = ((0, BASE_CHK),) if d <= 64 else ((0, 40), (40, BASE_CHK - 40))
  stage = max(n for _, n in passes)
  out_type = [jax.ShapeDtypeStruct((NC, NPAD, d), jnp.float32)]
  scratch = [
      pltpu.VMEM((stage, CHUNK), jnp.int32),
      pltpu.VMEM((stage, CHUNK), jnp.int32),
      pltpu.VMEM((sup * CHUNK, d), jnp.float32),
      pltpu.VMEM((sup * CHUNK, d), jnp.float32),
      pltpu.VMEM_SHARED((NPAD, d), jnp.float32),
      pltpu.SemaphoreType.DMA,
      pltpu.SemaphoreType.DMA,
      pltpu.SemaphoreType.DMA,
      pltpu.SemaphoreType.DMA,
  ]
  if with_deg:
    out_type.append(jax.ShapeDtypeStruct((NC, NPAD), jnp.float32))
    scratch += [
        pltpu.VMEM((CHUNK,), jnp.float32),
        pltpu.VMEM_SHARED((NPAD,), jnp.float32),
        pltpu.SemaphoreType.DMA,
    ]

  def body(table_hbm, zeros2d_hbm, zeros1d_hbm, ones_hbm, edges_hbm,
           part_hbm, *rest):
    if with_deg:
      (deg_hbm, src_v, dst_v, rows0_v, rows1_v, acc_sh, gsem0, gsem1,
       ssem0, ssem1, ones_v, deg_sh, dsem) = rest
    else:
      (src_v, dst_v, rows0_v, rows1_v, acc_sh, gsem0, gsem1,
       ssem0, ssem1) = rest
    c = lax.axis_index("c")
    s = lax.axis_index("s")
    w = c * NS + s
    base = BASE_CHK * w + lax.min(w, XTRA)
    r0 = s * ROWS_PER_TILE

    if with_deg:
      pltpu.sync_copy(ones_hbm, ones_v)
      pltpu.sync_copy(zeros1d_hbm.at[pl.ds(r0, ROWS_PER_TILE)],
                      deg_sh.at[pl.ds(r0, ROWS_PER_TILE)])
    pltpu.sync_copy(zeros2d_hbm.at[pl.ds(r0, ROWS_PER_TILE)],
                    acc_sh.at[pl.ds(r0, ROWS_PER_TILE)])
    plsc.subcore_barrier()

    def g_start(m, rows_v, gsem):
      for q in range(sup):
        pltpu.make_async_copy(table_hbm.at[src_v.at[sup * m + q]],
                              rows_v.at[pl.ds(q * CHUNK, CHUNK)],
                              gsem).start()

    def g_wait(m, rows_v, gsem):
      for q in range(sup):
        pltpu.make_async_copy(table_hbm.at[src_v.at[sup * m + q]],
                              rows_v.at[pl.ds(q * CHUNK, CHUNK)],
                              gsem).wait()

    def s_start(m, rows_v, ssem):
      for q in range(sup):
        pltpu.make_async_copy(rows_v.at[pl.ds(q * CHUNK, CHUNK)],
                              acc_sh.at[dst_v.at[sup * m + q]],
                              ssem).start(add=True)
        if with_deg:
          pltpu.make_async_copy(ones_v, deg_sh.at[dst_v.at[sup * m + q]],
                                dsem).start(add=True)

    def s_wait(m, rows_v, ssem):
      for q in range(sup):
        pltpu.make_async_copy(rows_v.at[pl.ds(q * CHUNK, CHUNK)],
                              acc_sh.at[dst_v.at[sup * m + q]],
                              ssem).wait()

    for off, npass in passes:
      pltpu.sync_copy(edges_hbm.at[0, pl.ds(base + off, npass)],
                      src_v.at[pl.ds(0, npass)])
      pltpu.sync_copy(edges_hbm.at[1, pl.ds(base + off, npass)],
                      dst_v.at[pl.ds(0, npass)])
      nsup = npass // sup
      nring = nsup // 2 * 2
      g_start(0, rows0_v, gsem0)
      g_start(1, rows1_v, gsem1)

      @pl.loop(0, nring, step=2)
      def _(j):
        j1 = lax.rem(j + 1, nring)
        j2 = lax.rem(j + 2, nring)
        j3 = lax.rem(j + 3, nring)
        g_wait(j, rows0_v, gsem0)
        s_start(j, rows0_v, ssem0)
        g_wait(j1, rows1_v, gsem1)
        s_start(j1, rows1_v, ssem1)
        s_wait(j, rows0_v, ssem0)
        g_start(j2, rows0_v, gsem0)
        s_wait(j1, rows1_v, ssem1)
        g_start(j3, rows1_v, gsem1)

      g_wait(0, rows0_v, gsem0)
      g_wait(1, rows1_v, gsem1)
      if nring < nsup:
        g_start(nring, rows0_v, gsem0)
        g_wait(nring, rows0_v, gsem0)
        s_start(nring, rows0_v, ssem0)
        s_wait(nring, rows0_v, ssem0)

      if with_deg:
        @pl.loop(0, npass)
        def _(j):
          pltpu.make_async_copy(ones_v, deg_sh.at[dst_v.at[0]], dsem).wait()

    @pl.when(w < XTRA)
    def _():
      xc = base + BASE_CHK
      pltpu.sync_copy(edges_hbm.at[0, xc], src_v.at[0])
      pltpu.sync_copy(edges_hbm.at[1, xc], dst_v.at[0])
      pltpu.make_async_copy(table_hbm.at[src_v.at[0]],
                            rows0_v.at[pl.ds(0, CHUNK)], gsem0).start()
      pltpu.make_async_copy(table_hbm.at[src_v.at[0]],
                            rows0_v.at[pl.ds(0, CHUNK)], gsem0).wait()
      pltpu.sync_copy(rows0_v.at[pl.ds(0, CHUNK)], acc_sh.at[dst_v.at[0]],
                      add=True)
      if with_deg:
        pltpu.sync_copy(ones_v, deg_sh.at[dst_v.at[0]], add=True)
    plsc.subcore_barrier()
    pltpu.sync_copy(acc_sh.at[pl.ds(r0, ROWS_PER_TILE)],
                    part_hbm.at[c, pl.ds(r0, ROWS_PER_TILE)])
    if with_deg:
      pltpu.sync_copy(deg_sh.at[pl.ds(r0, ROWS_PER_TILE)],
                      deg_hbm.at[c, pl.ds(r0, ROWS_PER_TILE)])

  return pl.kernel(body, out_type=out_type, mesh=mesh, scratch_types=scratch,
                   compiler_params=pltpu.CompilerParams(
                       use_tc_tiling_on_sc=False),
                   name=f"sc_segsum_d{d}")


_sc_agg_l0 = _make_sc_agg(D_IN, with_deg=True)
_sc_agg_l1 = _make_sc_agg(D2, with_deg=False)


def _tc1_body(x_ref, p_ref, degp_ref, ws0_ref, wn0_ref, b0_ref, wn1_ref,
              h_ref, z_ref):
  deg = jnp.maximum(degp_ref[0, :, 0] + degp_ref[1, :, 0], 1.0)
  hn = (p_ref[0] + p_ref[1]) / deg[:, None]
  h = (jnp.dot(x_ref[...], ws0_ref[...], preferred_element_type=jnp.float32)
       + jnp.dot(hn, wn0_ref[...], preferred_element_type=jnp.float32)
       + b0_ref[...])
  h = jnp.maximum(h, 0.0)
  h_ref[...] = h
  z_ref[...] = jnp.dot(h, wn1_ref[...], preferred_element_type=jnp.float32)


def _tc2_body(h_ref, q_ref, degp_ref, ws1_ref, b1_ref, out_ref):
  deg = jnp.maximum(degp_ref[0, :, 0] + degp_ref[1, :, 0], 1.0)
  hn = (q_ref[0] + q_ref[1]) / deg[:, None]
  out_ref[...] = (
      jnp.dot(h_ref[...], ws1_ref[...], preferred_element_type=jnp.float32)
      + hn + b1_ref[...])


_tc1 = pl.pallas_call(
    _tc1_body,
    grid=(NB,),
    in_specs=[
        pl.BlockSpec((BLK, D_IN), lambda i: (i, 0)),
        pl.BlockSpec((NC, BLK, D_IN), lambda i: (0, i, 0)),
        pl.BlockSpec((NC, BLK, 1), lambda i: (0, i, 0)),
        pl.BlockSpec((D_IN, D_HID), lambda i: (0, 0)),
        pl.BlockSpec((D_IN, D_HID), lambda i: (0, 0)),
        pl.BlockSpec((1, D_HID), lambda i: (0, 0)),
        pl.BlockSpec((D_HID, D2), lambda i: (0, 0)),
    ],
    out_specs=[
        pl.BlockSpec((BLK, D_HID), lambda i: (i, 0)),
        pl.BlockSpec((BLK, D2), lambda i: (i, 0)),
    ],
    out_shape=[
        jax.ShapeDtypeStruct((N_NODES, D_HID), jnp.float32),
        jax.ShapeDtypeStruct((N_NODES, D2), jnp.float32),
    ],
)

_tc2 = pl.pallas_call(
    _tc2_body,
    grid=(NB,),
    in_specs=[
        pl.BlockSpec((BLK, D_HID), lambda i: (i, 0)),
        pl.BlockSpec((NC, BLK, D2), lambda i: (0, i, 0)),
        pl.BlockSpec((NC, BLK, 1), lambda i: (0, i, 0)),
        pl.BlockSpec((D_HID, D2), lambda i: (0, 0)),
        pl.BlockSpec((1, D2), lambda i: (0, 0)),
    ],
    out_specs=pl.BlockSpec((BLK, D2), lambda i: (i, 0)),
    out_shape=jax.ShapeDtypeStruct((N_NODES, D2), jnp.float32),
)


@jax.jit
def kernel(x, edge_index, W_self0, W_neigh0, b0, W_self1, W_neigh1, b1):
  edges = edge_index.astype(jnp.int32).reshape(2, TOT_CHUNKS, CHUNK)

  zeros2d = jnp.zeros((NPAD, D_IN), jnp.float32)
  zeros2d_s = jnp.zeros((NPAD, D2), jnp.float32)
  zeros1d = jnp.zeros((NPAD,), jnp.float32)
  ones = jnp.ones((CHUNK,), jnp.float32)

  part0, degp = _sc_agg_l0(x, zeros2d, zeros1d, ones, edges)

  b0r = b0.reshape(1, D_HID)
  wn1p = jnp.pad(W_neigh1, ((0, 0), (0, D2 - N_CLASSES)))
  degp3 = degp.reshape(NC, NPAD, 1)
  h, z = _tc1(x, part0, degp3, W_self0, W_neigh0, b0r, wn1p)

  (part1,) = (_sc_agg_l1(z, zeros2d_s, zeros1d, ones, edges),)
  part1 = part1[0] if isinstance(part1, (list, tuple)) else part1

  ws1p = jnp.pad(W_self1, ((0, 0), (0, D2 - N_CLASSES)))
  b1p = jnp.pad(b1, (0, D2 - N_CLASSES)).reshape(1, D2)
  out = _tc2(h, part1, degp3, ws1p, b1p)
  return out[:, :N_CLASSES]


# speedup vs baseline: 1.0692x; 1.0122x over previous
import functools

import jax
import jax.numpy as jnp
import numpy as np
from jax import lax
from jax.experimental import pallas as pl
from jax.experimental.pallas import tpu as pltpu
from jax.experimental.pallas import tpu_sc as plsc

N_NODES = 10000
N_EDGES = 320000
D_IN = 128
D_HID = 128
N_CLASSES = 47

NC = 2
NS = 16
NW = NC * NS
CHUNK = 128
TOT_CHUNKS = N_EDGES // CHUNK
BASE_CHK = TOT_CHUNKS // NW
XTRA = TOT_CHUNKS - BASE_CHK * NW
NPAD = 10240
ROWS_PER_TILE = NPAD // NS
D2 = 48

BLK = 5000
NB = N_NODES // BLK


def _make_sc_agg(d, with_deg):
  mesh = plsc.VectorSubcoreMesh(core_axis_name="c", subcore_axis_name="s")
  sup = 2 if d <= 64 else 1
  passes = ((0, BASE_CHK),) if d <= 64 else ((0, 40), (40, BASE_CHK - 40))
  stage = max(n for _, n in passes)
  out_type = [jax.ShapeDtypeStruct((NC, NPAD, d), jnp.float32)]
  scratch = [
      pltpu.VMEM((stage, CHUNK), jnp.int32),
      pltpu.VMEM((stage, CHUNK), jnp.int32),
      pltpu.VMEM((sup * CHUNK, d), jnp.float32),
      pltpu.VMEM((sup * CHUNK, d), jnp.float32),
      pltpu.VMEM_SHARED((NPAD, d), jnp.float32),
      pltpu.SemaphoreType.DMA,
      pltpu.SemaphoreType.DMA,
      pltpu.SemaphoreType.DMA,
      pltpu.SemaphoreType.DMA,
  ]
  if with_deg:
    out_type.append(jax.ShapeDtypeStruct((NC, NPAD), jnp.float32))
    scratch += [
        pltpu.VMEM((CHUNK,), jnp.float32),
        pltpu.VMEM_SHARED((NPAD,), jnp.float32),
        pltpu.SemaphoreType.DMA,
    ]

  def body(table_hbm, zeros2d_hbm, zeros1d_hbm, ones_hbm, edges_hbm,
           part_hbm, *rest):
    if with_deg:
      (deg_hbm, src_v, dst_v, rows0_v, rows1_v, acc_sh, gsem0, gsem1,
       ssem0, ssem1, ones_v, deg_sh, dsem) = rest
    else:
      (src_v, dst_v, rows0_v, rows1_v, acc_sh, gsem0, gsem1,
       ssem0, ssem1) = rest
    c = lax.axis_index("c")
    s = lax.axis_index("s")
    w = c * NS + s
    base = BASE_CHK * w + lax.min((w + NS // 2 - 1) // (NS // 2), XTRA)
    r0 = s * ROWS_PER_TILE

    if with_deg:
      pltpu.sync_copy(ones_hbm, ones_v)
      pltpu.sync_copy(zeros1d_hbm.at[pl.ds(r0, ROWS_PER_TILE)],
                      deg_sh.at[pl.ds(r0, ROWS_PER_TILE)])
    pltpu.sync_copy(zeros2d_hbm.at[pl.ds(r0, ROWS_PER_TILE)],
                    acc_sh.at[pl.ds(r0, ROWS_PER_TILE)])
    plsc.subcore_barrier()

    def g_start(m, rows_v, gsem):
      for q in range(sup):
        pltpu.make_async_copy(table_hbm.at[src_v.at[sup * m + q]],
                              rows_v.at[pl.ds(q * CHUNK, CHUNK)],
                              gsem).start()

    def g_wait(m, rows_v, gsem):
      for q in range(sup):
        pltpu.make_async_copy(table_hbm.at[src_v.at[sup * m + q]],
                              rows_v.at[pl.ds(q * CHUNK, CHUNK)],
                              gsem).wait()

    def s_start(m, rows_v, ssem):
      for q in range(sup):
        pltpu.make_async_copy(rows_v.at[pl.ds(q * CHUNK, CHUNK)],
                              acc_sh.at[dst_v.at[sup * m + q]],
                              ssem).start(add=True)
        if with_deg:
          pltpu.make_async_copy(ones_v, deg_sh.at[dst_v.at[sup * m + q]],
                                dsem).start(add=True)

    def s_wait(m, rows_v, ssem):
      for q in range(sup):
        pltpu.make_async_copy(rows_v.at[pl.ds(q * CHUNK, CHUNK)],
                              acc_sh.at[dst_v.at[sup * m + q]],
                              ssem).wait()

    for off, npass in passes:
      pltpu.sync_copy(edges_hbm.at[0, pl.ds(base + off, npass)],
                      src_v.at[pl.ds(0, npass)])
      pltpu.sync_copy(edges_hbm.at[1, pl.ds(base + off, npass)],
                      dst_v.at[pl.ds(0, npass)])
      nsup = npass // sup
      nring = nsup // 2 * 2
      g_start(0, rows0_v, gsem0)
      g_start(1, rows1_v, gsem1)

      @pl.loop(0, nring, step=2)
      def _(j):
        j1 = lax.rem(j + 1, nring)
        j2 = lax.rem(j + 2, nring)
        j3 = lax.rem(j + 3, nring)
        g_wait(j, rows0_v, gsem0)
        s_start(j, rows0_v, ssem0)
        g_wait(j1, rows1_v, gsem1)
        s_start(j1, rows1_v, ssem1)
        s_wait(j, rows0_v, ssem0)
        g_start(j2, rows0_v, gsem0)
        s_wait(j1, rows1_v, ssem1)
        g_start(j3, rows1_v, gsem1)

      g_wait(0, rows0_v, gsem0)
      g_wait(1, rows1_v, gsem1)
      if nring < nsup:
        g_start(nring, rows0_v, gsem0)
        g_wait(nring, rows0_v, gsem0)
        s_start(nring, rows0_v, ssem0)
        s_wait(nring, rows0_v, ssem0)

      if with_deg:
        @pl.loop(0, npass)
        def _(j):
          pltpu.make_async_copy(ones_v, deg_sh.at[dst_v.at[0]], dsem).wait()

    @pl.when(lax.rem(w, NS // 2) == 0)
    def _():
      xc = base + BASE_CHK
      pltpu.sync_copy(edges_hbm.at[0, xc], src_v.at[0])
      pltpu.sync_copy(edges_hbm.at[1, xc], dst_v.at[0])
      pltpu.make_async_copy(table_hbm.at[src_v.at[0]],
                            rows0_v.at[pl.ds(0, CHUNK)], gsem0).start()
      pltpu.make_async_copy(table_hbm.at[src_v.at[0]],
                            rows0_v.at[pl.ds(0, CHUNK)], gsem0).wait()
      pltpu.sync_copy(rows0_v.at[pl.ds(0, CHUNK)], acc_sh.at[dst_v.at[0]],
                      add=True)
      if with_deg:
        pltpu.sync_copy(ones_v, deg_sh.at[dst_v.at[0]], add=True)
    plsc.subcore_barrier()
    pltpu.sync_copy(acc_sh.at[pl.ds(r0, ROWS_PER_TILE)],
                    part_hbm.at[c, pl.ds(r0, ROWS_PER_TILE)])
    if with_deg:
      pltpu.sync_copy(deg_sh.at[pl.ds(r0, ROWS_PER_TILE)],
                      deg_hbm.at[c, pl.ds(r0, ROWS_PER_TILE)])

  return pl.kernel(body, out_type=out_type, mesh=mesh, scratch_types=scratch,
                   compiler_params=pltpu.CompilerParams(
                       use_tc_tiling_on_sc=False),
                   name=f"sc_segsum_d{d}")


_sc_agg_l0 = _make_sc_agg(D_IN, with_deg=True)
_sc_agg_l1 = _make_sc_agg(D2, with_deg=False)


def _tc1_body(x_ref, p_ref, degp_ref, ws0_ref, wn0_ref, b0_ref, wn1_ref,
              h_ref, z_ref):
  deg = jnp.maximum(degp_ref[0, :, 0] + degp_ref[1, :, 0], 1.0)
  hn = (p_ref[0] + p_ref[1]) / deg[:, None]
  h = (jnp.dot(x_ref[...], ws0_ref[...], preferred_element_type=jnp.float32)
       + jnp.dot(hn, wn0_ref[...], preferred_element_type=jnp.float32)
       + b0_ref[...])
  h = jnp.maximum(h, 0.0)
  h_ref[...] = h
  z_ref[...] = jnp.dot(h, wn1_ref[...], preferred_element_type=jnp.float32)


def _tc2_body(h_ref, q_ref, degp_ref, ws1_ref, b1_ref, out_ref):
  deg = jnp.maximum(degp_ref[0, :, 0] + degp_ref[1, :, 0], 1.0)
  hn = (q_ref[0] + q_ref[1]) / deg[:, None]
  out_ref[...] = (
      jnp.dot(h_ref[...], ws1_ref[...], preferred_element_type=jnp.float32)
      + hn + b1_ref[...])


_tc1 = pl.pallas_call(
    _tc1_body,
    grid=(NB,),
    in_specs=[
        pl.BlockSpec((BLK, D_IN), lambda i: (i, 0)),
        pl.BlockSpec((NC, BLK, D_IN), lambda i: (0, i, 0)),
        pl.BlockSpec((NC, BLK, 1), lambda i: (0, i, 0)),
        pl.BlockSpec((D_IN, D_HID), lambda i: (0, 0)),
        pl.BlockSpec((D_IN, D_HID), lambda i: (0, 0)),
        pl.BlockSpec((1, D_HID), lambda i: (0, 0)),
        pl.BlockSpec((D_HID, D2), lambda i: (0, 0)),
    ],
    out_specs=[
        pl.BlockSpec((BLK, D_HID), lambda i: (i, 0)),
        pl.BlockSpec((BLK, D2), lambda i: (i, 0)),
    ],
    out_shape=[
        jax.ShapeDtypeStruct((N_NODES, D_HID), jnp.float32),
        jax.ShapeDtypeStruct((N_NODES, D2), jnp.float32),
    ],
)

_tc2 = pl.pallas_call(
    _tc2_body,
    grid=(NB,),
    in_specs=[
        pl.BlockSpec((BLK, D_HID), lambda i: (i, 0)),
        pl.BlockSpec((NC, BLK, D2), lambda i: (0, i, 0)),
        pl.BlockSpec((NC, BLK, 1), lambda i: (0, i, 0)),
        pl.BlockSpec((D_HID, D2), lambda i: (0, 0)),
        pl.BlockSpec((1, D2), lambda i: (0, 0)),
    ],
    out_specs=pl.BlockSpec((BLK, D2), lambda i: (i, 0)),
    out_shape=jax.ShapeDtypeStruct((N_NODES, D2), jnp.float32),
)


@jax.jit
def kernel(x, edge_index, W_self0, W_neigh0, b0, W_self1, W_neigh1, b1):
  edges = edge_index.astype(jnp.int32).reshape(2, TOT_CHUNKS, CHUNK)
  edges = jax.lax.optimization_barrier(edges)

  zeros2d = jnp.zeros((NPAD, D_IN), jnp.float32)
  zeros2d_s = jnp.zeros((NPAD, D2), jnp.float32)
  zeros1d = jnp.zeros((NPAD,), jnp.float32)
  ones = jnp.ones((CHUNK,), jnp.float32)

  part0, degp = _sc_agg_l0(x, zeros2d, zeros1d, ones, edges)

  b0r = b0.reshape(1, D_HID)
  wn1p = jnp.pad(W_neigh1, ((0, 0), (0, D2 - N_CLASSES)))
  degp3 = degp.reshape(NC, NPAD, 1)
  h, z = _tc1(x, part0, degp3, W_self0, W_neigh0, b0r, wn1p)

  (part1,) = (_sc_agg_l1(z, zeros2d_s, zeros1d, ones, edges),)
  part1 = part1[0] if isinstance(part1, (list, tuple)) else part1

  ws1p = jnp.pad(W_self1, ((0, 0), (0, D2 - N_CLASSES)))
  b1p = jnp.pad(b1, (0, D2 - N_CLASSES)).reshape(1, D2)
  out = _tc2(h, part1, degp3, ws1p, b1p)
  return out[:, :N_CLASSES]
